# Initial kernel scaffold; baseline (speedup 1.0000x reference)
import jax
import jax.numpy as jnp
from jax.experimental import pallas as pl


def kernel(sentence, context, W, neg_samples):
    raise NotImplementedError("write your pallas kernel here")


# speedup vs baseline: 3.6254x; 3.6254x over previous
import functools

import jax
import jax.numpy as jnp
from jax import lax
from jax.experimental import pallas as pl
from jax.experimental.pallas import tpu as pltpu
from jax.experimental.pallas import tpu_sc as plsc

_VOCAB = 1000
_EMBED = 128
_B = 1024
_L = 50
_NNEG = 5
_N = _B * _L
_NSAMP = _NNEG + 1
_NIDX = _NSAMP * _N

_NC = 2
_NS = 16
_NW = _NC * _NS
_PER_W = _NIDX // _NW
_CHUNK = 96
_NCHUNK = _PER_W // _CHUNK


def _sc_gather(table, idx):
    mesh = plsc.VectorSubcoreMesh(core_axis_name="c", subcore_axis_name="s")

    @functools.partial(
        pl.kernel,
        out_type=jax.ShapeDtypeStruct((_NIDX, _EMBED), jnp.float32),
        mesh=mesh,
        scratch_types=[
            pltpu.VMEM((_NCHUNK, _CHUNK), jnp.int32),
            pltpu.VMEM((2, _CHUNK, _EMBED), jnp.float32),
            pltpu.SemaphoreType.DMA,
            pltpu.SemaphoreType.DMA,
            pltpu.SemaphoreType.DMA,
        ],
    )
    def gather_kernel(table_hbm, idx_hbm, out_hbm, idx_v, rows_v, isem, gsem, osem):
        wid = lax.axis_index("s") * _NC + lax.axis_index("c")
        base = wid * _PER_W
        pltpu.async_copy(idx_hbm.at[wid], idx_v, isem).wait()

        @pl.loop(0, _NCHUNK)
        def _(g):
            pltpu.sync_copy(table_hbm.at[idx_v.at[g]], rows_v.at[0])
            pltpu.sync_copy(
                rows_v.at[0], out_hbm.at[pl.ds(base + g * _CHUNK, _CHUNK)]
            )

    return gather_kernel(table, idx)


def _logsig(x):
    return jnp.minimum(x, 0.0) - jnp.log1p(jnp.exp(-jnp.abs(x)))


_T = 512


def _tc_loss(emb, ctx):
    def body(e_ref, c_ref, o_ref):
        i = pl.program_id(0)
        c = c_ref[...]
        s_pos = jnp.sum(e_ref[0] * c, axis=1, keepdims=True)
        acc = _logsig(s_pos)
        for k in range(1, _NSAMP):
            s_neg = jnp.sum(e_ref[k] * c, axis=1, keepdims=True)
            acc = acc + _logsig(-s_neg)

        @pl.when(i == 0)
        def _():
            o_ref[0, 0] = 0.0

        o_ref[0, 0] += -jnp.sum(acc)

    out = pl.pallas_call(
        body,
        grid=(_N // _T,),
        in_specs=[
            pl.BlockSpec((_NSAMP, _T, _EMBED), lambda i: (0, i, 0)),
            pl.BlockSpec((_T, _EMBED), lambda i: (i, 0)),
        ],
        out_specs=pl.BlockSpec(memory_space=pltpu.SMEM),
        out_shape=jax.ShapeDtypeStruct((1, 1), jnp.float32),
    )(emb, ctx)
    return out[0, 0]


def kernel(sentence, context, W, neg_samples):
    idx = jnp.concatenate(
        [sentence.reshape(1, _N), neg_samples.reshape(_N, _NNEG).T], axis=0
    )
    idx = idx.reshape(_NW, _NCHUNK, _CHUNK).astype(jnp.int32)
    emb = _sc_gather(W, idx)
    ctx = context.reshape(_N, _EMBED)
    return _tc_loss(emb.reshape(_NSAMP, _N, _EMBED), ctx)


# speedup vs baseline: 3.8192x; 1.0535x over previous
import functools

import jax
import jax.numpy as jnp
from jax import lax
from jax.experimental import pallas as pl
from jax.experimental.pallas import tpu as pltpu
from jax.experimental.pallas import tpu_sc as plsc

_VOCAB = 1000
_EMBED = 128
_B = 1024
_L = 50
_NNEG = 5
_N = _B * _L
_NSAMP = _NNEG + 1
_NIDX = _NSAMP * _N

_NC = 2
_NS = 16
_NW = _NC * _NS
_PER_W = _NIDX // _NW
_CHUNK = 80
_NCHUNK = _PER_W // _CHUNK
_R = 4
_NGROUP = _NCHUNK // _R


def _sc_gather(table, idx):
    mesh = plsc.VectorSubcoreMesh(core_axis_name="c", subcore_axis_name="s")

    @functools.partial(
        pl.kernel,
        out_type=jax.ShapeDtypeStruct((_NIDX, _EMBED), jnp.float32),
        mesh=mesh,
        scratch_types=[
            pltpu.VMEM((_NCHUNK, _CHUNK), jnp.int32),
            pltpu.VMEM((2 * _R, _CHUNK, _EMBED), jnp.float32),
            pltpu.SemaphoreType.DMA,
            pltpu.SemaphoreType.DMA,
            pltpu.SemaphoreType.DMA,
            pltpu.SemaphoreType.DMA,
            pltpu.SemaphoreType.DMA,
        ],
    )
    def gather_kernel(
        table_hbm, idx_hbm, out_hbm, idx_v, rows_v, isem, gsem_a, gsem_b, osem_a, osem_b
    ):
        wid = lax.axis_index("s") * _NC + lax.axis_index("c")
        base = wid * _PER_W
        pltpu.async_copy(idx_hbm.at[wid], idx_v, isem).wait()

        def g_copy(gi, buf0, sem):
            return [
                pltpu.make_async_copy(
                    table_hbm.at[idx_v.at[gi * _R + b]], rows_v.at[buf0 + b], sem
                )
                for b in range(_R)
            ]

        def w_copy(gi, buf0, sem):
            return [
                pltpu.make_async_copy(
                    rows_v.at[buf0 + b],
                    out_hbm.at[pl.ds(base + (gi * _R + b) * _CHUNK, _CHUNK)],
                    sem,
                )
                for b in range(_R)
            ]

        def fire(copies):
            for c in copies:
                c.start()

        def drain(copies):
            for c in copies:
                c.wait()

        fire(g_copy(0, 0, gsem_a))

        @pl.loop(0, _NGROUP - 2, step=2)
        def _(gi):
            fire(g_copy(gi + 1, _R, gsem_b))
            drain(g_copy(gi, 0, gsem_a))
            fire(w_copy(gi, 0, osem_a))
            drain(g_copy(gi + 1, _R, gsem_b))
            drain(w_copy(gi, 0, osem_a))
            fire(g_copy(gi + 2, 0, gsem_a))
            fire(w_copy(gi + 1, _R, osem_b))
            drain(w_copy(gi + 1, _R, osem_b))

        gt = _NGROUP - 2
        fire(g_copy(gt + 1, _R, gsem_b))
        drain(g_copy(gt, 0, gsem_a))
        fire(w_copy(gt, 0, osem_a))
        drain(g_copy(gt + 1, _R, gsem_b))
        fire(w_copy(gt + 1, _R, osem_b))
        drain(w_copy(gt, 0, osem_a))
        drain(w_copy(gt + 1, _R, osem_b))

    return gather_kernel(table, idx)


def _logsig(x):
    return jnp.minimum(x, 0.0) - jnp.log1p(jnp.exp(-jnp.abs(x)))


_T = 512


def _tc_loss(emb, ctx):
    def body(e_ref, c_ref, o_ref):
        i = pl.program_id(0)
        c = c_ref[...]
        s_pos = jnp.sum(e_ref[0] * c, axis=1, keepdims=True)
        acc = _logsig(s_pos)
        for k in range(1, _NSAMP):
            s_neg = jnp.sum(e_ref[k] * c, axis=1, keepdims=True)
            acc = acc + _logsig(-s_neg)

        @pl.when(i == 0)
        def _():
            o_ref[0, 0] = 0.0

        o_ref[0, 0] += -jnp.sum(acc)

    out = pl.pallas_call(
        body,
        grid=(_N // _T,),
        in_specs=[
            pl.BlockSpec((_NSAMP, _T, _EMBED), lambda i: (0, i, 0)),
            pl.BlockSpec((_T, _EMBED), lambda i: (i, 0)),
        ],
        out_specs=pl.BlockSpec(memory_space=pltpu.SMEM),
        out_shape=jax.ShapeDtypeStruct((1, 1), jnp.float32),
    )(emb, ctx)
    return out[0, 0]


def kernel(sentence, context, W, neg_samples):
    idx = jnp.concatenate(
        [sentence.reshape(1, _N), neg_samples.reshape(_N, _NNEG).T], axis=0
    )
    idx = idx.reshape(_NW, _NCHUNK, _CHUNK).astype(jnp.int32)
    emb = _sc_gather(W, idx)
    ctx = context.reshape(_N, _EMBED)
    return _tc_loss(emb.reshape(_NSAMP, _N, _EMBED), ctx)
